# baseline (speedup 1.0000x reference)
import functools

import jax
import jax.numpy as jnp
from jax import lax
from jax.experimental import pallas as pl
from jax.experimental.pallas import tpu as pltpu
from jax.experimental.pallas import tpu_sc as plsc

_BATCH = 4096
_T = 200
_D = 64
_NC = 2
_NS = 16
_NW = _NC * _NS
_BB = _BATCH // _NW
_NG = 4
_NTR = 2

_mesh = plsc.VectorSubcoreMesh(core_axis_name="c", subcore_axis_name="s")


@functools.partial(
    pl.kernel,
    mesh=_mesh,
    out_type=jax.ShapeDtypeStruct((_T, 8, 32, 8, 128), jnp.float32),
    scratch_types=[
        pltpu.VMEM((_T, _BB), jnp.int32),
        pltpu.VMEM((_NG, _BB, _D), jnp.float32),
        pltpu.VMEM((_NTR, 8, 1, 8, 128), jnp.float32),
        pltpu.SemaphoreType.DMA((_NG,)),
        pltpu.SemaphoreType.DMA((_NTR,)),
    ],
    compiler_params=pltpu.CompilerParams(use_tc_tiling_on_sc=False, needs_layout_passes=False),
)
def _gather(table_hbm, idx_hbm, out_hbm, idx_v, g_v, tr_v, gsem, ssem):
    wid = lax.axis_index("s") * _NC + lax.axis_index("c")

    pltpu.sync_copy(idx_hbm.at[wid], idx_v)

    base16 = lax.iota(jnp.int32, 16)
    bvecs = [base16 + 16 * k for k in range(8)]

    def start_gather(t, r):
        pltpu.async_copy(table_hbm.at[idx_v.at[t]], g_v.at[r], gsem.at[r])

    def wait_gather(r):
        pltpu.make_async_copy(table_hbm.at[idx_v.at[0]], g_v.at[r], gsem.at[r]).wait()

    def start_store(t, s):
        pltpu.async_copy(
            tr_v.at[s], out_hbm.at[t, pl.ds(0, 8), pl.ds(wid, 1)], ssem.at[s]
        )

    def wait_store(s):
        pltpu.make_async_copy(
            tr_v.at[s], out_hbm.at[0, pl.ds(0, 8), pl.ds(0, 1)], ssem.at[s]
        ).wait()

    def transpose(r, s):
        @plsc.parallel_loop(0, _D, unroll=8)
        def _d(d):
            db = lax.div(d, 8)
            di = lax.rem(d, 8)
            dcol = jnp.full((16,), 0, jnp.int32) + d
            for k in range(8):
                val = plsc.load_gather(g_v.at[r], [bvecs[k], dcol])
                tr_v[s, db, 0, di, pl.ds(16 * k, 16)] = val

    def step(t, r, s, do_wait_store, do_refill):
        wait_gather(r)
        if do_wait_store:
            wait_store(s)
        transpose(r, s)
        if do_refill:
            start_gather(t + _NG, r)
        start_store(t, s)

    for t in range(_NG):
        start_gather(t, t)

    for t in range(_NG):
        step(t, t, t % _NTR, t >= _NTR, True)

    @pl.loop(_NG, _T - _NG, step=_NG)
    def _body(tbase):
        for b in range(_NG):
            step(tbase + b, b, b % _NTR, True, True)

    for t in range(_T - _NG, _T):
        step(t, t % _NG, t % _NTR, True, False)

    for s in range(_NTR):
        wait_store(s)


def kernel(x, position_embedding):
    idx = x.reshape(_NW, _BB, _T).transpose(0, 2, 1)
    out5 = _gather(position_embedding, idx)
    out = out5.transpose(0, 1, 3, 2, 4)
    out = out.reshape(_T, _D, _BATCH)
    return out.transpose(2, 0, 1)


# speedup vs baseline: 1.0756x; 1.0756x over previous
import functools

import jax
import jax.numpy as jnp
from jax import lax
from jax.experimental import pallas as pl
from jax.experimental.pallas import tpu as pltpu
from jax.experimental.pallas import tpu_sc as plsc

_B = 4096 * 200
_D = 64
_NC = 2
_NS = 16
_NW = _NC * _NS
_R = _B // _NW
_C = 128
_STEPS = _R // _C
_NBUF = 4

_mesh = plsc.VectorSubcoreMesh(core_axis_name="c", subcore_axis_name="s")


@functools.partial(
    pl.kernel,
    mesh=_mesh,
    out_type=jax.ShapeDtypeStruct((_B, _D), jnp.float32),
    scratch_types=[
        pltpu.VMEM((_STEPS, _C), jnp.int32),
        pltpu.VMEM((_NBUF, _C, _D), jnp.float32),
        pltpu.SemaphoreType.DMA((_NBUF,)),
        pltpu.SemaphoreType.DMA((_NBUF,)),
    ],
    compiler_params=pltpu.CompilerParams(use_tc_tiling_on_sc=False),
)
def _gather(table_hbm, idx_hbm, out_hbm, idx_v, rows_v, gsem, ssem):
    wid = lax.axis_index("s") * _NC + lax.axis_index("c")
    cbase = wid * _STEPS

    pltpu.sync_copy(idx_hbm.at[pl.ds(cbase, _STEPS)], idx_v)

    def start_gather(g, b):
        return pltpu.async_copy(table_hbm.at[idx_v.at[g]], rows_v.at[b], gsem.at[b])

    def wait_gather(b):
        pltpu.make_async_copy(table_hbm.at[idx_v.at[0]], rows_v.at[b], gsem.at[b]).wait()

    def start_store(g, b):
        return pltpu.async_copy(rows_v.at[b], out_hbm.at[pl.ds((cbase + g) * _C, _C)], ssem.at[b])

    def wait_store(b):
        pltpu.make_async_copy(rows_v.at[b], out_hbm.at[pl.ds(0, _C)], ssem.at[b]).wait()

    for b in range(_NBUF):
        start_gather(b, b)

    @pl.loop(0, _STEPS - _NBUF, step=_NBUF)
    def _body(gbase):
        for b in range(_NBUF):
            g = gbase + b
            wait_gather(b)
            start_store(g, b)
            wait_store(b)
            start_gather(g + _NBUF, b)

    for b in range(_NBUF):
        wait_gather(b)
        start_store(_STEPS - _NBUF + b, b)
    for b in range(_NBUF):
        wait_store(b)


def kernel(x, position_embedding):
    flat = x.reshape(-1, _C)
    out = _gather(position_embedding, flat)
    return out.reshape(x.shape + (position_embedding.shape[1],))
